# baseline (speedup 1.0000x reference)
import functools

import jax
import jax.numpy as jnp
from jax import lax
from jax.experimental import pallas as pl
from jax.experimental.pallas import tpu as pltpu
from jax.experimental.pallas import tpu_sc as plsc

BATCH = 16384
NUM_FACTORS = 32
HIDDEN = 128
_IDX_CHUNK = 128


def _make_sc_gather(B, D):
    info = plsc.get_sparse_core_info()
    NC, NS = info.num_cores, info.num_subcores
    NW = NC * NS
    b_per_w = B // NW
    n_chunks = b_per_w // _IDX_CHUNK
    mesh = plsc.VectorSubcoreMesh(core_axis_name="c", subcore_axis_name="s")

    @functools.partial(
        pl.kernel,
        out_type=(
            jax.ShapeDtypeStruct((B, D), jnp.float32),
            jax.ShapeDtypeStruct((B, D), jnp.float32),
        ),
        mesh=mesh,
        compiler_params=pltpu.CompilerParams(use_tc_tiling_on_sc=False),
        scratch_types=[
            pltpu.VMEM((n_chunks, _IDX_CHUNK), jnp.int32),
            pltpu.VMEM((n_chunks, _IDX_CHUNK), jnp.int32),
            pltpu.VMEM((b_per_w, D), jnp.float32),
            pltpu.VMEM((b_per_w, D), jnp.float32),
            pltpu.SemaphoreType.DMA,
        ],
    )
    def sc_gather(uid_hbm, iid_hbm, utab_hbm, itab_hbm, uout_hbm, iout_hbm,
                  uidx_v, iidx_v, urows_v, irows_v, sem):
        wid = lax.axis_index("s") * NC + lax.axis_index("c")
        base = wid * b_per_w
        for j in range(n_chunks):
            pltpu.sync_copy(uid_hbm.at[pl.ds(base + j * _IDX_CHUNK, _IDX_CHUNK)],
                            uidx_v.at[j])
            pltpu.sync_copy(iid_hbm.at[pl.ds(base + j * _IDX_CHUNK, _IDX_CHUNK)],
                            iidx_v.at[j])
        copies = []
        for j in range(n_chunks):
            copies.append(pltpu.async_copy(
                utab_hbm.at[uidx_v.at[j]],
                urows_v.at[pl.ds(j * _IDX_CHUNK, _IDX_CHUNK)], sem))
            copies.append(pltpu.async_copy(
                itab_hbm.at[iidx_v.at[j]],
                irows_v.at[pl.ds(j * _IDX_CHUNK, _IDX_CHUNK)], sem))
        for c in copies:
            c.wait()
        pltpu.sync_copy(urows_v, uout_hbm.at[pl.ds(base, b_per_w)])
        pltpu.sync_copy(irows_v, iout_hbm.at[pl.ds(base, b_per_w)])

    return sc_gather


def _mlp_body(u_ref, i_ref, e_ref, w1u_ref, w1i_ref, w1e_ref, b1_ref,
              w2_ref, b2_ref, o_ref):
    cdims = (((1,), (1,)), ((), ()))
    h = lax.dot_general(u_ref[...], w1u_ref[...], cdims,
                        preferred_element_type=jnp.float32)
    h = h + lax.dot_general(i_ref[...], w1i_ref[...], cdims,
                            preferred_element_type=jnp.float32)
    h = h + e_ref[...] * w1e_ref[...] + b1_ref[...]
    h = jnp.maximum(h, 0.0)
    o_ref[...] = jnp.sum(h * w2_ref[...], axis=1, keepdims=True) + b2_ref[0, 0]


def _make_tc_mlp(B, D, H, blk):
    grid = B // blk
    const = lambda *_: (0, 0)
    return pl.pallas_call(
        _mlp_body,
        grid=(grid,),
        in_specs=[
            pl.BlockSpec((blk, D), lambda i: (i, 0)),
            pl.BlockSpec((blk, D), lambda i: (i, 0)),
            pl.BlockSpec((blk, 1), lambda i: (i, 0)),
            pl.BlockSpec((H, D), const),
            pl.BlockSpec((H, D), const),
            pl.BlockSpec((1, H), const),
            pl.BlockSpec((1, H), const),
            pl.BlockSpec((1, H), const),
            pl.BlockSpec((1, 1), const),
        ],
        out_specs=pl.BlockSpec((blk, 1), lambda i: (i, 0)),
        out_shape=jax.ShapeDtypeStruct((B, 1), jnp.float32),
    )


def kernel(user_ids, item_ids, exposures_hat, user_table, item_table,
           W1, b1, W2, b2):
    B = user_ids.shape[0]
    D = user_table.shape[1]
    H = W1.shape[0]
    urows, irows = _make_sc_gather(B, D)(user_ids, item_ids,
                                         user_table, item_table)
    w1u = W1[:, :D]
    w1i = W1[:, D:2 * D]
    w1e = W1[:, 2 * D].reshape(1, H)
    out = _make_tc_mlp(B, D, H, blk=2048)(
        urows, irows, exposures_hat.reshape(B, 1),
        w1u, w1i, w1e, b1.reshape(1, H), W2, b2.reshape(1, 1))
    return out[:, 0]


# speedup vs baseline: 3.3209x; 3.3209x over previous
import functools

import jax
import jax.numpy as jnp
from jax import lax
from jax.experimental import pallas as pl
from jax.experimental.pallas import tpu as pltpu
from jax.experimental.pallas import tpu_sc as plsc

_CH = 16
_TW = 128


def _make_sc_gather(B, D, V):
    info = plsc.get_sparse_core_info()
    NC, NS = info.num_cores, info.num_subcores
    NW = NC * NS
    b_per_w = B // NW
    n_ch = b_per_w // _CH
    mesh = plsc.VectorSubcoreMesh(core_axis_name="c", subcore_axis_name="s")

    @functools.partial(
        pl.kernel,
        out_type=(
            jax.ShapeDtypeStruct((D, B), jnp.float32),
            jax.ShapeDtypeStruct((D, B), jnp.float32),
        ),
        mesh=mesh,
        compiler_params=pltpu.CompilerParams(needs_layout_passes=False),
        scratch_types=[
            pltpu.VMEM((_CH,), jnp.int32),
            pltpu.VMEM((_CH, D, _TW), jnp.float32),
            pltpu.VMEM((D, b_per_w), jnp.float32),
            pltpu.SemaphoreType.DMA,
        ],
    )
    def sc_gather(uid_hbm, iid_hbm, utT_hbm, itT_hbm, uoutT_hbm, ioutT_hbm,
                  idv, win, outv, sem):
        wid = lax.axis_index("s") * NC + lax.axis_index("c")
        base = wid * b_per_w

        def do_table(ids_hbm, tT_hbm, outT_hbm):
            def chunk_body(ch):
                pltpu.sync_copy(ids_hbm.at[pl.ds(base + ch * _CH, _CH)], idv)
                rvec = (idv[...] >> 7) * _TW
                lanes = lax.iota(jnp.int32, 16)
                copies = []
                for i in range(_CH):
                    r0 = jnp.max(jnp.where(lanes == i, rvec, 0))
                    r0 = pl.multiple_of(r0, _TW)
                    copies.append(pltpu.async_copy(
                        tT_hbm.at[:, pl.ds(r0, _TW)], win.at[i], sem))
                for c in copies:
                    c.wait()
                for g in range(_CH // 16):
                    i_vec = lax.iota(jnp.int32, 16) + g * 16
                    roff_vec = idv[pl.ds(g * 16, 16)] & (_TW - 1)
                    for c in range(D):
                        c_vec = jnp.full((16,), c, jnp.int32)
                        vals = plsc.load_gather(win, [i_vec, c_vec, roff_vec])
                        outv[c, pl.ds(ch * _CH + g * 16, 16)] = vals
            pl.loop(0, n_ch)(chunk_body)
            pltpu.sync_copy(outv, outT_hbm.at[:, pl.ds(base, b_per_w)])

        do_table(uid_hbm, utT_hbm, uoutT_hbm)
        do_table(iid_hbm, itT_hbm, ioutT_hbm)

    return sc_gather


def _mlp_body(u_ref, i_ref, e_ref, w1ui_ref, web_ref, w2_ref, b2_ref, o_ref):
    x2 = jnp.concatenate([u_ref[...], i_ref[...]], axis=0)
    eb = jnp.concatenate(
        [e_ref[...], jnp.ones_like(e_ref[...])], axis=0)
    h = lax.dot_general(w1ui_ref[...], x2, (((1,), (0,)), ((), ())),
                        preferred_element_type=jnp.float32)
    h = h + lax.dot_general(web_ref[...], eb, (((1,), (0,)), ((), ())),
                            preferred_element_type=jnp.float32)
    h = jnp.maximum(h, 0.0)
    o_ref[...] = lax.dot_general(w2_ref[...], h, (((1,), (0,)), ((), ())),
                                 preferred_element_type=jnp.float32) + b2_ref[0, 0]


def _make_tc_mlp(B, D, H, blk):
    const = lambda *_: (0, 0)
    return pl.pallas_call(
        _mlp_body,
        grid=(B // blk,),
        in_specs=[
            pl.BlockSpec((D, blk), lambda i: (0, i)),
            pl.BlockSpec((D, blk), lambda i: (0, i)),
            pl.BlockSpec((1, blk), lambda i: (0, i)),
            pl.BlockSpec((H, 2 * D), const),
            pl.BlockSpec((H, 2), const),
            pl.BlockSpec((1, H), const),
            pl.BlockSpec((1, 1), const),
        ],
        out_specs=pl.BlockSpec((1, blk), lambda i: (0, i)),
        out_shape=jax.ShapeDtypeStruct((1, B), jnp.float32),
    )


def kernel(user_ids, item_ids, exposures_hat, user_table, item_table,
           W1, b1, W2, b2):
    B = user_ids.shape[0]
    V, D = user_table.shape
    H = W1.shape[0]
    uT, iT = _make_sc_gather(B, D, V)(user_ids, item_ids,
                                      user_table.T, item_table.T)
    w1ui = W1[:, :2 * D]
    web = jnp.concatenate([W1[:, 2 * D:2 * D + 1], b1[:, None]], axis=1)
    out = _make_tc_mlp(B, D, H, blk=2048)(
        uT, iT, exposures_hat.reshape(1, B), w1ui, web, W2, b2.reshape(1, 1))
    return out.reshape(B)


# speedup vs baseline: 3.6346x; 1.0945x over previous
import functools

import jax
import jax.numpy as jnp
from jax import lax
from jax.experimental import pallas as pl
from jax.experimental.pallas import tpu as pltpu
from jax.experimental.pallas import tpu_sc as plsc

_CH = 16
_TW = 128


def _make_sc_gather(B, D, V):
    info = plsc.get_sparse_core_info()
    NC, NS = info.num_cores, info.num_subcores
    NW = NC * NS
    b_per_w = B // NW
    n_ch = b_per_w // _CH
    mesh = plsc.VectorSubcoreMesh(core_axis_name="c", subcore_axis_name="s")

    @functools.partial(
        pl.kernel,
        out_type=(
            jax.ShapeDtypeStruct((D, B), jnp.float32),
            jax.ShapeDtypeStruct((D, B), jnp.float32),
        ),
        mesh=mesh,
        compiler_params=pltpu.CompilerParams(needs_layout_passes=False),
        scratch_types=[
            pltpu.VMEM((512,), jnp.int32),
            pltpu.VMEM((_CH, D, _TW), jnp.float32),
            pltpu.VMEM((D, b_per_w), jnp.float32),
            pltpu.SemaphoreType.DMA,
        ],
    )
    def sc_gather(uid_hbm, iid_hbm, utT_hbm, itT_hbm, uoutT_hbm, ioutT_hbm,
                  idv, win, outv, sem):
        wid = lax.axis_index("s") * NC + lax.axis_index("c")
        base = wid * b_per_w

        lanes = lax.iota(jnp.int32, 16)

        def do_table(ids_hbm, tT_hbm, outT_hbm):
            pltpu.sync_copy(ids_hbm.at[pl.ds(base, b_per_w)], idv)

            def chunk_body(ch):
                idc = idv[pl.ds(ch * _CH, 16)]
                rvec = (idc >> 7) * _TW
                copies = []
                for i in range(_CH):
                    r0 = jnp.max(jnp.where(lanes == i, rvec, 0))
                    r0 = pl.multiple_of(r0, _TW)
                    copies.append(pltpu.async_copy(
                        tT_hbm.at[:, pl.ds(r0, _TW)], win.at[i], sem))
                for c in copies:
                    c.wait()
                roff_vec = idc & (_TW - 1)
                for c in range(D):
                    c_vec = jnp.full((16,), c, jnp.int32)
                    vals = plsc.load_gather(win, [lanes, c_vec, roff_vec])
                    outv[c, pl.ds(ch * _CH, 16)] = vals
            pl.loop(0, n_ch)(chunk_body)
            pltpu.sync_copy(outv, outT_hbm.at[:, pl.ds(base, b_per_w)])

        do_table(uid_hbm, utT_hbm, uoutT_hbm)
        do_table(iid_hbm, itT_hbm, ioutT_hbm)

    return sc_gather


def _mlp_body(u_ref, i_ref, e_ref, w1ui_ref, web_ref, w2_ref, b2_ref, o_ref):
    x2 = jnp.concatenate([u_ref[...], i_ref[...]], axis=0)
    eb = jnp.concatenate(
        [e_ref[...], jnp.ones_like(e_ref[...])], axis=0)
    h = lax.dot_general(w1ui_ref[...], x2, (((1,), (0,)), ((), ())),
                        preferred_element_type=jnp.float32)
    h = h + lax.dot_general(web_ref[...], eb, (((1,), (0,)), ((), ())),
                            preferred_element_type=jnp.float32)
    h = jnp.maximum(h, 0.0)
    o_ref[...] = lax.dot_general(w2_ref[...], h, (((1,), (0,)), ((), ())),
                                 preferred_element_type=jnp.float32) + b2_ref[0, 0]


def _make_tc_mlp(B, D, H, blk):
    const = lambda *_: (0, 0)
    return pl.pallas_call(
        _mlp_body,
        grid=(B // blk,),
        in_specs=[
            pl.BlockSpec((D, blk), lambda i: (0, i)),
            pl.BlockSpec((D, blk), lambda i: (0, i)),
            pl.BlockSpec((1, blk), lambda i: (0, i)),
            pl.BlockSpec((H, 2 * D), const),
            pl.BlockSpec((H, 2), const),
            pl.BlockSpec((1, H), const),
            pl.BlockSpec((1, 1), const),
        ],
        out_specs=pl.BlockSpec((1, blk), lambda i: (0, i)),
        out_shape=jax.ShapeDtypeStruct((1, B), jnp.float32),
    )


def kernel(user_ids, item_ids, exposures_hat, user_table, item_table,
           W1, b1, W2, b2):
    B = user_ids.shape[0]
    V, D = user_table.shape
    H = W1.shape[0]
    uT, iT = _make_sc_gather(B, D, V)(user_ids, item_ids,
                                      user_table.T, item_table.T)
    w1ui = W1[:, :2 * D]
    web = jnp.concatenate([W1[:, 2 * D:2 * D + 1], b1[:, None]], axis=1)
    out = _make_tc_mlp(B, D, H, blk=2048)(
        uT, iT, exposures_hat.reshape(1, B), w1ui, web, W2, b2.reshape(1, 1))
    return out.reshape(B)


# speedup vs baseline: 3.6382x; 1.0010x over previous
import functools

import jax
import jax.numpy as jnp
from jax import lax
from jax.experimental import pallas as pl
from jax.experimental.pallas import tpu as pltpu
from jax.experimental.pallas import tpu_sc as plsc

_H = 8
_TW = 128


def _make_sc_gather(B, D, V):
    info = plsc.get_sparse_core_info()
    NC, NS = info.num_cores, info.num_subcores
    NW = NC * NS
    b_per_w = B // NW
    n_hc = b_per_w // _H
    mesh = plsc.VectorSubcoreMesh(core_axis_name="c", subcore_axis_name="s")

    @functools.partial(
        pl.kernel,
        out_type=(
            jax.ShapeDtypeStruct((D, B), jnp.float32),
            jax.ShapeDtypeStruct((D, B), jnp.float32),
        ),
        mesh=mesh,
        compiler_params=pltpu.CompilerParams(needs_layout_passes=False),
        scratch_types=[
            pltpu.VMEM((b_per_w + 16, ), jnp.int32),
            pltpu.VMEM((2, _H, D, _TW), jnp.float32),
            pltpu.VMEM((D, b_per_w), jnp.float32),
            pltpu.SemaphoreType.DMA,
            pltpu.SemaphoreType.DMA,
        ],
    )
    def sc_gather(uid_hbm, iid_hbm, utT_hbm, itT_hbm, uoutT_hbm, ioutT_hbm,
                  idv, win, outv, sem0, sem1):
        wid = lax.axis_index("s") * NC + lax.axis_index("c")
        base = wid * b_per_w

        lanes = lax.iota(jnp.int32, 16)
        sems = (sem0, sem1)

        def do_table(ids_hbm, tT_hbm, outT_hbm):
            pltpu.sync_copy(ids_hbm.at[pl.ds(base, b_per_w)],
                            idv.at[pl.ds(0, b_per_w)])

            def issue(j, par):
                idc = idv[pl.ds(j * _H, 16)]
                rvec = (idc >> 7) * _TW
                for i in range(_H):
                    r0 = jnp.max(jnp.where(lanes == i, rvec, 0))
                    r0 = pl.multiple_of(r0, _TW)
                    pltpu.async_copy(
                        tT_hbm.at[:, pl.ds(r0, _TW)], win.at[par, i],
                        sems[par])

            def drain(par):
                for i in range(_H):
                    pltpu.make_async_copy(
                        tT_hbm.at[:, pl.ds(0, _TW)], win.at[par, i],
                        sems[par]).wait()

            def extract(j, par):
                idc = idv[pl.ds(j * _H, 16)]
                roff_vec = idc & (_TW - 1)
                mask = lanes < _H
                pos = j * _H + lanes
                for c in range(D):
                    c_vec = jnp.full((16,), c, jnp.int32)
                    vals = plsc.load_gather(win.at[par],
                                            [lanes, c_vec, roff_vec],
                                            mask=mask)
                    plsc.store_scatter(outv, [c_vec, pos], vals, mask=mask)

            issue(0, 0)

            def pair_body(jj):
                j0 = jj * 2
                issue(j0 + 1, 1)
                drain(0)
                extract(j0, 0)
                pl.when(j0 + 2 < n_hc)(lambda: issue(j0 + 2, 0))
                drain(1)
                extract(j0 + 1, 1)

            pl.loop(0, n_hc // 2)(pair_body)
            pltpu.sync_copy(outv, outT_hbm.at[:, pl.ds(base, b_per_w)])

        do_table(uid_hbm, utT_hbm, uoutT_hbm)
        do_table(iid_hbm, itT_hbm, ioutT_hbm)

    return sc_gather


def _mlp_body(u_ref, i_ref, e_ref, w1ui_ref, web_ref, w2_ref, b2_ref, o_ref):
    x2 = jnp.concatenate([u_ref[...], i_ref[...]], axis=0)
    eb = jnp.concatenate(
        [e_ref[...], jnp.ones_like(e_ref[...])], axis=0)
    h = lax.dot_general(w1ui_ref[...], x2, (((1,), (0,)), ((), ())),
                        preferred_element_type=jnp.float32)
    h = h + lax.dot_general(web_ref[...], eb, (((1,), (0,)), ((), ())),
                            preferred_element_type=jnp.float32)
    h = jnp.maximum(h, 0.0)
    o_ref[...] = lax.dot_general(w2_ref[...], h, (((1,), (0,)), ((), ())),
                                 preferred_element_type=jnp.float32) + b2_ref[0, 0]


def _make_tc_mlp(B, D, H, blk):
    const = lambda *_: (0, 0)
    return pl.pallas_call(
        _mlp_body,
        grid=(B // blk,),
        in_specs=[
            pl.BlockSpec((D, blk), lambda i: (0, i)),
            pl.BlockSpec((D, blk), lambda i: (0, i)),
            pl.BlockSpec((1, blk), lambda i: (0, i)),
            pl.BlockSpec((H, 2 * D), const),
            pl.BlockSpec((H, 2), const),
            pl.BlockSpec((1, H), const),
            pl.BlockSpec((1, 1), const),
        ],
        out_specs=pl.BlockSpec((1, blk), lambda i: (0, i)),
        out_shape=jax.ShapeDtypeStruct((1, B), jnp.float32),
    )


def kernel(user_ids, item_ids, exposures_hat, user_table, item_table,
           W1, b1, W2, b2):
    B = user_ids.shape[0]
    V, D = user_table.shape
    H = W1.shape[0]
    uT, iT = _make_sc_gather(B, D, V)(user_ids, item_ids,
                                      user_table.T, item_table.T)
    w1ui = W1[:, :2 * D]
    web = jnp.concatenate([W1[:, 2 * D:2 * D + 1], b1[:, None]], axis=1)
    out = _make_tc_mlp(B, D, H, blk=2048)(
        uT, iT, exposures_hat.reshape(1, B), w1ui, web, W2, b2.reshape(1, 1))
    return out.reshape(B)


# speedup vs baseline: 3.6823x; 1.0121x over previous
import functools

import jax
import jax.numpy as jnp
from jax import lax
from jax.experimental import pallas as pl
from jax.experimental.pallas import tpu as pltpu
from jax.experimental.pallas import tpu_sc as plsc

_H = 8
_TW = 128


def _make_sc_gather(B, D, V):
    info = plsc.get_sparse_core_info()
    NC, NS = info.num_cores, info.num_subcores
    NW = NC * NS
    b_per_w = B // NW
    n_hc = b_per_w // _H
    mesh = plsc.VectorSubcoreMesh(core_axis_name="c", subcore_axis_name="s")

    @functools.partial(
        pl.kernel,
        out_type=(
            jax.ShapeDtypeStruct((D, B), jnp.float32),
            jax.ShapeDtypeStruct((D, B), jnp.float32),
        ),
        mesh=mesh,
        compiler_params=pltpu.CompilerParams(needs_layout_passes=False),
        scratch_types=[
            pltpu.VMEM((b_per_w + 16, ), jnp.int32),
            pltpu.VMEM((2, _H, D, _TW), jnp.float32),
            pltpu.VMEM((D, b_per_w), jnp.float32),
            pltpu.SemaphoreType.DMA,
            pltpu.SemaphoreType.DMA,
        ],
    )
    def sc_gather(uid_hbm, iid_hbm, utT_hbm, itT_hbm, uoutT_hbm, ioutT_hbm,
                  idv, win, outv, sem0, sem1):
        wid = lax.axis_index("s") * NC + lax.axis_index("c")
        base = wid * b_per_w

        lanes = lax.iota(jnp.int32, 16)
        sems = (sem0, sem1)

        def do_table(ids_hbm, tT_hbm, outT_hbm):
            pltpu.sync_copy(ids_hbm.at[pl.ds(base, b_per_w)],
                            idv.at[pl.ds(0, b_per_w)])

            def issue(j, par):
                idc = idv[pl.ds(j * _H, 16)]
                rvec = (idc >> 7) * _TW
                for i in range(_H):
                    r0 = jnp.max(jnp.where(lanes == i, rvec, 0))
                    r0 = pl.multiple_of(r0, _TW)
                    pltpu.async_copy(
                        tT_hbm.at[:, pl.ds(r0, _TW)], win.at[par, i],
                        sems[par])

            def drain(par):
                for i in range(_H):
                    pltpu.make_async_copy(
                        tT_hbm.at[:, pl.ds(0, _TW)], win.at[par, i],
                        sems[par]).wait()

            def extract(j, par):
                idc = idv[pl.ds(j * _H, 16)]
                roff_vec = idc & (_TW - 1)
                mask = lanes < _H
                pos = j * _H + lanes
                for c in range(D):
                    c_vec = jnp.full((16,), c, jnp.int32)
                    vals = plsc.load_gather(win.at[par],
                                            [lanes, c_vec, roff_vec],
                                            mask=mask)
                    plsc.store_scatter(outv, [c_vec, pos], vals, mask=mask)

            issue(0, 0)

            def pair_body(jj):
                j0 = jj * 2
                issue(j0 + 1, 1)
                drain(0)
                extract(j0, 0)
                pl.when(j0 + 2 < n_hc)(lambda: issue(j0 + 2, 0))
                drain(1)
                extract(j0 + 1, 1)

            pl.loop(0, n_hc // 2)(pair_body)
            pltpu.sync_copy(outv, outT_hbm.at[:, pl.ds(base, b_per_w)])

        do_table(uid_hbm, utT_hbm, uoutT_hbm)
        do_table(iid_hbm, itT_hbm, ioutT_hbm)

    return sc_gather


def _mlp_body(u_ref, i_ref, e_ref, w1ui_ref, web_ref, w2_ref, b2_ref, o_ref):
    x2 = jnp.concatenate([u_ref[...], i_ref[...]], axis=0)
    eb = jnp.concatenate(
        [e_ref[...], jnp.ones_like(e_ref[...])], axis=0)
    h = lax.dot_general(w1ui_ref[...], x2, (((1,), (0,)), ((), ())),
                        preferred_element_type=jnp.float32)
    h = h + lax.dot_general(web_ref[...], eb, (((1,), (0,)), ((), ())),
                            preferred_element_type=jnp.float32)
    h = jnp.maximum(h, 0.0)
    o_ref[...] = lax.dot_general(w2_ref[...], h, (((1,), (0,)), ((), ())),
                                 preferred_element_type=jnp.float32) + b2_ref[0, 0]


def _make_tc_mlp(B, D, H, blk):
    const = lambda *_: (0, 0)
    return pl.pallas_call(
        _mlp_body,
        grid=(B // blk,),
        in_specs=[
            pl.BlockSpec((D, blk), lambda i: (0, i)),
            pl.BlockSpec((D, blk), lambda i: (0, i)),
            pl.BlockSpec((1, blk), lambda i: (0, i)),
            pl.BlockSpec((H, 2 * D), const),
            pl.BlockSpec((H, 2), const),
            pl.BlockSpec((1, H), const),
            pl.BlockSpec((1, 1), const),
        ],
        out_specs=pl.BlockSpec((1, blk), lambda i: (0, i)),
        out_shape=jax.ShapeDtypeStruct((1, B), jnp.float32),
    )


def kernel(user_ids, item_ids, exposures_hat, user_table, item_table,
           W1, b1, W2, b2):
    B = user_ids.shape[0]
    V, D = user_table.shape
    H = W1.shape[0]
    uT, iT = _make_sc_gather(B, D, V)(user_ids, item_ids,
                                      user_table.T, item_table.T)
    w1ui = W1[:, :2 * D]
    web = jnp.concatenate([W1[:, 2 * D:2 * D + 1], b1[:, None]], axis=1)
    out = _make_tc_mlp(B, D, H, blk=8192)(
        uT, iT, exposures_hat.reshape(1, B), w1ui, web, W2, b2.reshape(1, 1))
    return out.reshape(B)


# speedup vs baseline: 3.6867x; 1.0012x over previous
import functools

import jax
import jax.numpy as jnp
from jax import lax
from jax.experimental import pallas as pl
from jax.experimental.pallas import tpu as pltpu
from jax.experimental.pallas import tpu_sc as plsc

_H = 8
_TW = 128


def _make_sc_gather(B, D, V):
    info = plsc.get_sparse_core_info()
    NC, NS = info.num_cores, info.num_subcores
    NW = NC * NS
    b_per_w = B // NW
    n_hc = b_per_w // _H
    mesh = plsc.VectorSubcoreMesh(core_axis_name="c", subcore_axis_name="s")

    @functools.partial(
        pl.kernel,
        out_type=(
            jax.ShapeDtypeStruct((D, B), jnp.float32),
            jax.ShapeDtypeStruct((D, B), jnp.float32),
        ),
        mesh=mesh,
        compiler_params=pltpu.CompilerParams(needs_layout_passes=False),
        scratch_types=[
            pltpu.VMEM((b_per_w + 16, ), jnp.int32),
            pltpu.VMEM((2, _H, D, _TW), jnp.float32),
            pltpu.VMEM((D, b_per_w), jnp.float32),
            pltpu.SemaphoreType.DMA,
            pltpu.SemaphoreType.DMA,
        ],
    )
    def sc_gather(uid_hbm, iid_hbm, utT_hbm, itT_hbm, uoutT_hbm, ioutT_hbm,
                  idv, win, outv, sem0, sem1):
        wid = lax.axis_index("s") * NC + lax.axis_index("c")
        base = wid * b_per_w

        lanes = lax.iota(jnp.int32, 16)
        sems = (sem0, sem1)

        def do_table(ids_hbm, tT_hbm, outT_hbm):
            pltpu.sync_copy(ids_hbm.at[pl.ds(base, b_per_w)],
                            idv.at[pl.ds(0, b_per_w)])

            def issue(j, par):
                idc = idv[pl.ds(j * _H, 16)]
                rvec = (idc >> 7) * _TW
                for i in range(_H):
                    r0 = jnp.max(jnp.where(lanes == i, rvec, 0))
                    r0 = pl.multiple_of(r0, _TW)
                    pltpu.async_copy(
                        tT_hbm.at[:, pl.ds(r0, _TW)], win.at[par, i],
                        sems[par])

            def drain(par):
                for i in range(_H):
                    pltpu.make_async_copy(
                        tT_hbm.at[:, pl.ds(0, _TW)], win.at[par, i],
                        sems[par]).wait()

            def extract(j, par):
                idc = idv[pl.ds(j * _H, 16)]
                roff_vec = idc & (_TW - 1)
                mask = lanes < _H
                pos = j * _H + lanes
                for c in range(D):
                    c_vec = jnp.full((16,), c, jnp.int32)
                    vals = plsc.load_gather(win.at[par],
                                            [lanes, c_vec, roff_vec],
                                            mask=mask)
                    plsc.store_scatter(outv, [c_vec, pos], vals, mask=mask)

            issue(0, 0)

            def pair_body(jj):
                j0 = jj * 2
                issue(j0 + 1, 1)
                drain(0)
                extract(j0, 0)
                pl.when(j0 + 2 < n_hc)(lambda: issue(j0 + 2, 0))
                drain(1)
                extract(j0 + 1, 1)

            pl.loop(0, n_hc // 2)(pair_body)
            pltpu.sync_copy(outv, outT_hbm.at[:, pl.ds(base, b_per_w)])

        do_table(uid_hbm, utT_hbm, uoutT_hbm)
        do_table(iid_hbm, itT_hbm, ioutT_hbm)

    return sc_gather


def _mlp_body(u_ref, i_ref, e_ref, w1ui_ref, web_ref, w2_ref, b2_ref, o_ref):
    x2 = jnp.concatenate([u_ref[...], i_ref[...]], axis=0)
    eb = jnp.concatenate(
        [e_ref[...], jnp.ones_like(e_ref[...])], axis=0)
    h = lax.dot_general(w1ui_ref[...], x2, (((1,), (0,)), ((), ())),
                        preferred_element_type=jnp.float32)
    h = h + lax.dot_general(web_ref[...], eb, (((1,), (0,)), ((), ())),
                            preferred_element_type=jnp.float32)
    h = jnp.maximum(h, 0.0)
    o_ref[...] = lax.dot_general(w2_ref[...], h, (((1,), (0,)), ((), ())),
                                 preferred_element_type=jnp.float32) + b2_ref[0, 0]


def _make_tc_mlp(B, D, H, blk):
    const = lambda *_: (0, 0)
    return pl.pallas_call(
        _mlp_body,
        grid=(B // blk,),
        in_specs=[
            pl.BlockSpec((D, blk), lambda i: (0, i)),
            pl.BlockSpec((D, blk), lambda i: (0, i)),
            pl.BlockSpec((1, blk), lambda i: (0, i)),
            pl.BlockSpec((H, 2 * D), const),
            pl.BlockSpec((H, 2), const),
            pl.BlockSpec((1, H), const),
            pl.BlockSpec((1, 1), const),
        ],
        out_specs=pl.BlockSpec((1, blk), lambda i: (0, i)),
        out_shape=jax.ShapeDtypeStruct((1, B), jnp.float32),
    )


def kernel(user_ids, item_ids, exposures_hat, user_table, item_table,
           W1, b1, W2, b2):
    B = user_ids.shape[0]
    V, D = user_table.shape
    H = W1.shape[0]
    uT, iT = _make_sc_gather(B, D, V)(user_ids, item_ids,
                                      user_table.T, item_table.T)
    w1ui = W1[:, :2 * D]
    web = jnp.concatenate([W1[:, 2 * D:2 * D + 1], b1[:, None]], axis=1)
    out = _make_tc_mlp(B, D, H, blk=16384)(
        uT, iT, exposures_hat.reshape(1, B), w1ui, web, W2, b2.reshape(1, 1))
    return out.reshape(B)
